# Initial kernel scaffold; baseline (speedup 1.0000x reference)
import jax
import jax.numpy as jnp
from jax.experimental import pallas as pl


def kernel(edge_type, table):
    raise NotImplementedError("write your pallas kernel here")


# speedup vs baseline: 3.5371x; 3.5371x over previous
import functools

import jax
import jax.numpy as jnp
from jax import lax
from jax.experimental import pallas as pl
from jax.experimental.pallas import tpu as pltpu
from jax.experimental.pallas import tpu_sc as plsc

_D = 64
_B = 4096 * 200
_info = plsc.get_sparse_core_info()
_NC = _info.num_cores
_NS = _info.num_subcores
_NW = _NC * _NS
_IDXW = 128
_BPW = _B // _NW
_NCH = _BPW // _IDXW

_mesh = plsc.VectorSubcoreMesh(core_axis_name="c", subcore_axis_name="s")


@functools.partial(
    pl.kernel,
    mesh=_mesh,
    out_type=jax.ShapeDtypeStruct((_B, _D), jnp.float32),
    scratch_types=[
        pltpu.VMEM((_NCH, _IDXW), jnp.int32),
        pltpu.VMEM((_IDXW, _D), jnp.float32),
        pltpu.SemaphoreType.DMA,
    ],
    compiler_params=pltpu.CompilerParams(use_tc_tiling_on_sc=False),
)
def _gather(table_hbm, idx_hbm, out_hbm, idx_v, rows_v, gsem):
    wid = lax.axis_index("s") * _NC + lax.axis_index("c")
    pltpu.sync_copy(idx_hbm.at[pl.ds(wid * _NCH, _NCH)], idx_v)

    def body(c, carry):
        pltpu.async_copy(table_hbm.at[idx_v.at[c]], rows_v, gsem).wait()
        pltpu.sync_copy(rows_v,
                        out_hbm.at[pl.ds((wid * _NCH + c) * _IDXW, _IDXW)])
        return carry

    lax.fori_loop(0, _NCH, body, 0)


def kernel(edge_type, table):
    idx = edge_type.reshape(-1).astype(jnp.int32).reshape(_NW * _NCH, _IDXW)
    out = _gather(table, idx)
    return out.reshape(edge_type.shape + (_D,))


# speedup vs baseline: 4.2182x; 1.1925x over previous
import functools

import jax
import jax.numpy as jnp
from jax import lax
from jax.experimental import pallas as pl
from jax.experimental.pallas import tpu as pltpu
from jax.experimental.pallas import tpu_sc as plsc

_D = 64
_B = 4096 * 200
_info = plsc.get_sparse_core_info()
_NC = _info.num_cores
_NS = _info.num_subcores
_NW = _NC * _NS
_IDXW = 128
_BPW = _B // _NW
_NCH = _BPW // _IDXW
_K = 4
_C = _K * _IDXW
_NCHK = _BPW // _C
_NBUF = 2
_NGRP = _NCHK // _NBUF

_mesh = plsc.VectorSubcoreMesh(core_axis_name="c", subcore_axis_name="s")


@functools.partial(
    pl.kernel,
    mesh=_mesh,
    out_type=jax.ShapeDtypeStruct((_B, _D), jnp.float32),
    scratch_types=[
        pltpu.VMEM((_NCH, _IDXW), jnp.int32),
        pltpu.VMEM((_C, _D), jnp.float32),
        pltpu.VMEM((_C, _D), jnp.float32),
        pltpu.SemaphoreType.DMA,
        pltpu.SemaphoreType.DMA,
        pltpu.SemaphoreType.DMA,
        pltpu.SemaphoreType.DMA,
    ],
    compiler_params=pltpu.CompilerParams(use_tc_tiling_on_sc=False),
)
def _gather(table_hbm, idx_hbm, out_hbm, idx_v, rows0, rows1, g0, g1, s0, s1):
    wid = lax.axis_index("s") * _NC + lax.axis_index("c")
    pltpu.sync_copy(idx_hbm.at[pl.ds(wid * _NCH, _NCH)], idx_v)
    bufs = ((rows0, g0, s0), (rows1, g1, s1))

    def fire_gathers(c, buf, gsem):
        for j in range(_K):
            pltpu.async_copy(table_hbm.at[idx_v.at[c * _K + j]],
                             buf.at[pl.ds(j * _IDXW, _IDXW)], gsem)

    def drain_gathers(buf, gsem):
        pltpu.make_async_copy(table_hbm.at[pl.ds(0, _C)], buf, gsem).wait()

    def drain_store(buf, ssem):
        pltpu.make_async_copy(buf, out_hbm.at[pl.ds(0, _C)], ssem).wait()

    for b in range(_NBUF):
        fire_gathers(b, bufs[b][0], bufs[b][1])

    def body(i, carry):
        for b in range(_NBUF):
            c = i * _NBUF + b
            buf, gsem, ssem = bufs[b]
            drain_gathers(buf, gsem)
            pltpu.async_copy(buf, out_hbm.at[pl.ds(wid * _BPW + c * _C, _C)],
                             ssem)

        @pl.when(i < _NGRP - 1)
        def _prefetch():
            for b in range(_NBUF):
                buf, gsem, ssem = bufs[b]
                drain_store(buf, ssem)
                fire_gathers((i + 1) * _NBUF + b, buf, gsem)

        return carry

    lax.fori_loop(0, _NGRP, body, 0)
    for b in range(_NBUF):
        drain_store(bufs[b][0], bufs[b][2])


def kernel(edge_type, table):
    idx = edge_type.reshape(-1).astype(jnp.int32).reshape(_NW * _NCH, _IDXW)
    out = _gather(table, idx)
    return out.reshape(edge_type.shape + (_D,))


# speedup vs baseline: 4.2405x; 1.0053x over previous
import functools

import jax
import jax.numpy as jnp
from jax import lax
from jax.experimental import pallas as pl
from jax.experimental.pallas import tpu as pltpu
from jax.experimental.pallas import tpu_sc as plsc

_D = 64
_B = 4096 * 200
_info = plsc.get_sparse_core_info()
_NC = _info.num_cores
_NS = _info.num_subcores
_NW = _NC * _NS
_IDXW = 128
_BPW = _B // _NW
_NCH = _BPW // _IDXW
_K = 2
_C = _K * _IDXW
_NCHK = _BPW // _C
_NBUF = 4
_NGRP = _NCHK // _NBUF
assert _NCHK % _NBUF == 0

_mesh = plsc.VectorSubcoreMesh(core_axis_name="c", subcore_axis_name="s")


@functools.partial(
    pl.kernel,
    mesh=_mesh,
    out_type=jax.ShapeDtypeStruct((_B, _D), jnp.float32),
    scratch_types=(
        [pltpu.VMEM((_NCH, _IDXW), jnp.int32)]
        + [pltpu.VMEM((_C, _D), jnp.float32) for _ in range(_NBUF)]
        + [pltpu.SemaphoreType.DMA for _ in range(2 * _NBUF)]
    ),
    compiler_params=pltpu.CompilerParams(use_tc_tiling_on_sc=False),
)
def _gather(table_hbm, idx_hbm, out_hbm, idx_v, *scratch):
    rows = scratch[:_NBUF]
    gsems = scratch[_NBUF:2 * _NBUF]
    ssems = scratch[2 * _NBUF:]
    wid = lax.axis_index("s") * _NC + lax.axis_index("c")
    pltpu.sync_copy(idx_hbm.at[pl.ds(wid * _NCH, _NCH)], idx_v)

    def fire_gathers(c, b):
        for j in range(_K):
            pltpu.async_copy(table_hbm.at[idx_v.at[c * _K + j]],
                             rows[b].at[pl.ds(j * _IDXW, _IDXW)], gsems[b])

    def drain_gathers(b):
        pltpu.make_async_copy(table_hbm.at[pl.ds(0, _C)], rows[b],
                              gsems[b]).wait()

    def drain_store(b):
        pltpu.make_async_copy(rows[b], out_hbm.at[pl.ds(0, _C)],
                              ssems[b]).wait()

    for b in range(_NBUF):
        fire_gathers(b, b)

    def body(i, carry):
        for b in range(_NBUF):
            c = i * _NBUF + b
            drain_gathers(b)
            pltpu.async_copy(rows[b],
                             out_hbm.at[pl.ds(wid * _BPW + c * _C, _C)],
                             ssems[b])

        @pl.when(i < _NGRP - 1)
        def _prefetch():
            for b in range(_NBUF):
                drain_store(b)
                fire_gathers((i + 1) * _NBUF + b, b)

        return carry

    lax.fori_loop(0, _NGRP, body, 0)
    for b in range(_NBUF):
        drain_store(b)


def kernel(edge_type, table):
    idx = edge_type.reshape(-1).astype(jnp.int32).reshape(_NW * _NCH, _IDXW)
    out = _gather(table, idx)
    return out.reshape(edge_type.shape + (_D,))


# speedup vs baseline: 4.2436x; 1.0007x over previous
import functools

import jax
import jax.numpy as jnp
from jax import lax
from jax.experimental import pallas as pl
from jax.experimental.pallas import tpu as pltpu
from jax.experimental.pallas import tpu_sc as plsc

_D = 64
_B0 = 4096
_T = 200
_info = plsc.get_sparse_core_info()
_NC = _info.num_cores
_NS = _info.num_subcores
_NW = _NC * _NS
_GPW = _B0 // _NW
_SPLITS = ((0, 96), (96, 104))
_NBUF = 4
_NGRP = _GPW // _NBUF
assert _GPW % _NBUF == 0

_mesh = plsc.VectorSubcoreMesh(core_axis_name="c", subcore_axis_name="s")


@functools.partial(
    pl.kernel,
    mesh=_mesh,
    out_type=jax.ShapeDtypeStruct((_B0, _T, _D), jnp.float32),
    scratch_types=(
        [pltpu.VMEM((_GPW, _T), jnp.int32)]
        + [pltpu.VMEM((_T, _D), jnp.float32) for _ in range(_NBUF)]
        + [pltpu.SemaphoreType.DMA for _ in range(2 * _NBUF)]
    ),
    compiler_params=pltpu.CompilerParams(use_tc_tiling_on_sc=False),
)
def _gather(table_hbm, idx_hbm, out_hbm, idx_v, *scratch):
    rows = scratch[:_NBUF]
    gsems = scratch[_NBUF:2 * _NBUF]
    ssems = scratch[2 * _NBUF:]
    wid = lax.axis_index("s") * _NC + lax.axis_index("c")
    pltpu.sync_copy(idx_hbm.at[pl.ds(wid * _GPW, _GPW)], idx_v)

    def fire_gathers(c, b):
        for off, ln in _SPLITS:
            pltpu.async_copy(table_hbm.at[idx_v.at[c, pl.ds(off, ln)]],
                             rows[b].at[pl.ds(off, ln)], gsems[b])

    def drain_gathers(b):
        pltpu.make_async_copy(table_hbm.at[pl.ds(0, _T)], rows[b],
                              gsems[b]).wait()

    def drain_store(b):
        pltpu.make_async_copy(rows[b], out_hbm.at[0], ssems[b]).wait()

    for b in range(_NBUF):
        fire_gathers(b, b)

    def body(i, carry):
        for b in range(_NBUF):
            c = i * _NBUF + b
            drain_gathers(b)
            pltpu.async_copy(rows[b], out_hbm.at[wid * _GPW + c], ssems[b])

        @pl.when(i < _NGRP - 1)
        def _prefetch():
            for b in range(_NBUF):
                drain_store(b)
                fire_gathers((i + 1) * _NBUF + b, b)

        return carry

    lax.fori_loop(0, _NGRP, body, 0)
    for b in range(_NBUF):
        drain_store(b)


def kernel(edge_type, table):
    return _gather(table, edge_type.astype(jnp.int32))
